# Initial kernel scaffold; baseline (speedup 1.0000x reference)
import jax
import jax.numpy as jnp
from jax.experimental import pallas as pl


def kernel(x, edge_index, edge_attr, W1, b1, W2, b2, W3, b3):
    raise NotImplementedError("write your pallas kernel here")


# speedup vs baseline: 9.3995x; 9.3995x over previous
import functools

import jax
import jax.numpy as jnp
from jax import lax
from jax.experimental import pallas as pl
from jax.experimental.pallas import tpu as pltpu
from jax.experimental.pallas import tpu_sc as plsc

NC = 2
NS = 16
L = 16
NW = NC * NS
CH = 128


def _mesh():
    return plsc.VectorSubcoreMesh(core_axis_name="c", subcore_axis_name="s")


_SC_PARAMS = pltpu.CompilerParams(use_tc_tiling_on_sc=False)


def _pad_nodes(n_nodes):
    per = -(-n_nodes // (NS * 40)) * 40
    return per * NS


def _zero_acc_slice(zb, acc, sid, n_rows_per_subcore):
    zr, d = zb.shape

    @pl.loop(0, zr)
    def _(r):
        for f in range(d // L):
            zb[r, pl.ds(f * L, L)] = jnp.zeros((L,), jnp.float32)

    @pl.loop(0, n_rows_per_subcore // zr)
    def _(z):
        pltpu.sync_copy(zb, acc.at[pl.ds(sid * n_rows_per_subcore + z * zr, zr)])


def _sc_degree(col3, ewb, n_nodes):
    c_chunks = col3.shape[1]
    n_pad = _pad_nodes(n_nodes)
    rps = n_pad // NS

    @functools.partial(
        pl.kernel,
        out_type=jax.ShapeDtypeStruct((NC, n_pad, L), jnp.float32),
        mesh=_mesh(),
        compiler_params=_SC_PARAMS,
        scratch_types=[
            pltpu.VMEM((c_chunks, CH), jnp.int32),
            pltpu.VMEM((CH, L), jnp.float32),
            pltpu.VMEM_SHARED((n_pad, L), jnp.float32),
        ],
    )
    def k(col_hbm, ewb_hbm, out_hbm, col_v, ewb_v, acc):
        cid = lax.axis_index("c")
        sid = lax.axis_index("s")
        wid = sid * NC + cid
        pltpu.sync_copy(col_hbm.at[wid], col_v)
        _zero_acc_slice(ewb_v, acc, sid, rps)
        plsc.subcore_barrier()

        @pl.loop(0, c_chunks)
        def _(c):
            pltpu.sync_copy(ewb_hbm.at[wid, c], ewb_v)
            pltpu.sync_copy(ewb_v, acc.at[col_v.at[c]], add=True)

        plsc.subcore_barrier()
        pltpu.sync_copy(acc.at[pl.ds(sid * rps, rps)],
                        out_hbm.at[cid, pl.ds(sid * rps, rps)])

    return k(col3, ewb)


def _sc_scatter(g, row3, col3, ewb, n_nodes):
    d = g.shape[1]
    c_chunks = row3.shape[1]
    n_pad = _pad_nodes(n_nodes)
    rps = n_pad // NS
    fd = d // L

    @functools.partial(
        pl.kernel,
        out_type=jax.ShapeDtypeStruct((NC, n_pad, d), jnp.float32),
        mesh=_mesh(),
        compiler_params=_SC_PARAMS,
        scratch_types=[
            pltpu.VMEM((c_chunks, CH), jnp.int32),
            pltpu.VMEM((c_chunks, CH), jnp.int32),
            pltpu.VMEM((CH, L), jnp.float32),
            pltpu.VMEM((CH, d), jnp.float32),
            pltpu.VMEM_SHARED((n_pad, d), jnp.float32),
        ],
    )
    def k(g_hbm, row_hbm, col_hbm, ewb_hbm, out_hbm,
          row_v, col_v, ewb_v, gbuf, acc):
        cid = lax.axis_index("c")
        sid = lax.axis_index("s")
        wid = sid * NC + cid
        pltpu.sync_copy(row_hbm.at[wid], row_v)
        pltpu.sync_copy(col_hbm.at[wid], col_v)
        _zero_acc_slice(gbuf, acc, sid, rps)
        plsc.subcore_barrier()

        @pl.loop(0, c_chunks)
        def _(c):
            pltpu.sync_copy(ewb_hbm.at[wid, c], ewb_v)
            pltpu.sync_copy(g_hbm.at[row_v.at[c]], gbuf)

            @pl.loop(0, CH)
            def _(j):
                s = ewb_v[j, :]
                for f in range(fd):
                    sl = pl.ds(f * L, L)
                    gbuf[j, sl] = gbuf[j, sl] * s

            pltpu.sync_copy(gbuf, acc.at[col_v.at[c]], add=True)

        plsc.subcore_barrier()
        pltpu.sync_copy(acc.at[pl.ds(sid * rps, rps)],
                        out_hbm.at[cid, pl.ds(sid * rps, rps)])

    return k(g, row3, col3, ewb)


def _dis_from(degp_blk):
    deg = degp_blk[0, :, 0] + degp_blk[1, :, 0] + 1.0
    return jnp.where(deg > 0, lax.rsqrt(jnp.maximum(deg, 1e-30)), 0.0)


_B = 1000


def _tc_pre(degp, x, w):
    n, d_in = x.shape
    d_out = w.shape[1]

    def body(degp_ref, x_ref, w_ref, g_ref):
        dis = _dis_from(degp_ref[...])
        h = jnp.dot(x_ref[...], w_ref[...], preferred_element_type=jnp.float32)
        g_ref[...] = dis[:, None] * h

    return pl.pallas_call(
        body,
        grid=(n // _B,),
        in_specs=[
            pl.BlockSpec((NC, _B, L), lambda i: (0, i, 0)),
            pl.BlockSpec((_B, d_in), lambda i: (i, 0)),
            pl.BlockSpec((d_in, d_out), lambda i: (0, 0)),
        ],
        out_specs=pl.BlockSpec((_B, d_out), lambda i: (i, 0)),
        out_shape=jax.ShapeDtypeStruct((n, d_out), jnp.float32),
    )(degp, x, w)


def _tc_mid(degp, p, g, b, w):
    n, d = g.shape
    d_out = w.shape[1]

    def body(degp_ref, p_ref, g_ref, b_ref, w_ref, gn_ref):
        dis = _dis_from(degp_ref[...])
        s = p_ref[0] + p_ref[1] + g_ref[...]
        o = dis[:, None] * s + b_ref[...]
        a = jnp.where(o > 0, o, jnp.exp(o) - 1.0)
        h = jnp.dot(a, w_ref[...], preferred_element_type=jnp.float32)
        gn_ref[...] = dis[:, None] * h

    return pl.pallas_call(
        body,
        grid=(n // _B,),
        in_specs=[
            pl.BlockSpec((NC, _B, L), lambda i: (0, i, 0)),
            pl.BlockSpec((NC, _B, d), lambda i: (0, i, 0)),
            pl.BlockSpec((_B, d), lambda i: (i, 0)),
            pl.BlockSpec((1, d), lambda i: (0, 0)),
            pl.BlockSpec((d, d_out), lambda i: (0, 0)),
        ],
        out_specs=pl.BlockSpec((_B, d_out), lambda i: (i, 0)),
        out_shape=jax.ShapeDtypeStruct((n, d_out), jnp.float32),
    )(degp, p, g, b.reshape(1, d), w)


def _tc_final(degp, p, g, b):
    n, d = g.shape

    def body(degp_ref, p_ref, g_ref, b_ref, o_ref):
        dis = _dis_from(degp_ref[...])
        s = p_ref[0] + p_ref[1] + g_ref[...]
        o_ref[...] = dis[:, None] * s + b_ref[...]

    return pl.pallas_call(
        body,
        grid=(n // _B,),
        in_specs=[
            pl.BlockSpec((NC, _B, L), lambda i: (0, i, 0)),
            pl.BlockSpec((NC, _B, d), lambda i: (0, i, 0)),
            pl.BlockSpec((_B, d), lambda i: (i, 0)),
            pl.BlockSpec((1, d), lambda i: (0, 0)),
        ],
        out_specs=pl.BlockSpec((_B, d), lambda i: (i, 0)),
        out_shape=jax.ShapeDtypeStruct((n, d), jnp.float32),
    )(degp, p, g, b.reshape(1, d))


def kernel(x, edge_index, edge_attr, W1, b1, W2, b2, W3, b3):
    n = x.shape[0]
    e = edge_index.shape[1]
    c_chunks = -(-e // (NW * CH))
    e_pad = NW * CH * c_chunks
    pad = e_pad - e

    row3 = jnp.pad(edge_index[0], (0, pad)).reshape(NW, c_chunks, CH)
    col3 = jnp.pad(edge_index[1], (0, pad)).reshape(NW, c_chunks, CH)
    ew = jnp.pad(edge_attr, (0, pad)).reshape(NW, c_chunks, CH)
    ewb = jnp.broadcast_to(ew[..., None], (NW, c_chunks, CH, L)) + 0.0

    degp = _sc_degree(col3, ewb, n)
    g1 = _tc_pre(degp, x, W1)
    p1 = _sc_scatter(g1, row3, col3, ewb, n)
    g2 = _tc_mid(degp, p1, g1, b1, W2)
    p2 = _sc_scatter(g2, row3, col3, ewb, n)
    g3 = _tc_mid(degp, p2, g2, b2, W3)
    p3 = _sc_scatter(g3, row3, col3, ewb, n)
    return _tc_final(degp, p3, g3, b3)
